# baseline (speedup 1.0000x reference)
import functools

import jax
import jax.numpy as jnp
from jax import lax
from jax.experimental import pallas as pl
from jax.experimental.pallas import tpu as pltpu
from jax.experimental.pallas import tpu_sc as plsc


_NEG = -1e30


def _sc_gather(K_flat, V_flat, gidx, d, ch=32):
    n = gidx.shape[0]
    info = plsc.get_sparse_core_info()
    nw = info.num_cores * info.num_subcores
    per_w = n // nw
    mesh = plsc.VectorSubcoreMesh(core_axis_name="c", subcore_axis_name="s")
    out = jax.ShapeDtypeStruct((n, d), jnp.float32)

    @functools.partial(
        pl.kernel, mesh=mesh,
        out_type=[out, out],
        scratch_types=[
            pltpu.VMEM((per_w,), jnp.int32),
            pltpu.VMEM((ch, d), jnp.float32),
            pltpu.VMEM((ch, d), jnp.float32),
            pltpu.SemaphoreType.DMA,
            pltpu.SemaphoreType.DMA,
        ],
    )
    def k(kt_hbm, vt_hbm, gidx_hbm, ko_hbm, vo_hbm,
          idx_v, krows_v, vrows_v, ksem, vsem):
        wid = lax.axis_index("s") * info.num_cores + lax.axis_index("c")
        base = wid * per_w
        pltpu.sync_copy(gidx_hbm.at[pl.ds(base, per_w)], idx_v)
        for c in range(per_w // ch):
            kcp = pltpu.async_copy(
                kt_hbm.at[idx_v.at[pl.ds(c * ch, ch)]], krows_v, ksem)
            vcp = pltpu.async_copy(
                vt_hbm.at[idx_v.at[pl.ds(c * ch, ch)]], vrows_v, vsem)
            kcp.wait()
            pltpu.sync_copy(krows_v, ko_hbm.at[pl.ds(base + c * ch, ch)])
            vcp.wait()
            pltpu.sync_copy(vrows_v, vo_hbm.at[pl.ds(base + c * ch, ch)])

    return k(K_flat, V_flat, gidx)


def _qprobe_body(h_ref, q_ref):
    q_ref[...] = jnp.mean(h_ref[...], axis=1, keepdims=True)


def _qprobe(h):
    b, seq, d = h.shape
    return pl.pallas_call(
        _qprobe_body,
        grid=(b,),
        in_specs=[pl.BlockSpec((1, seq, d), lambda i: (i, 0, 0))],
        out_specs=pl.BlockSpec((1, 1, d), lambda i: (i, 0, 0)),
        out_shape=jax.ShapeDtypeStruct((b, 1, d), jnp.float32),
        compiler_params=pltpu.CompilerParams(
            dimension_semantics=("arbitrary",)),
    )(h)


def _surprise_body(q_ref, ka_ref, va_ref, ks_ref, vs_ref, *, scale):
    q = q_ref[0]
    for (c_ref, s_ref) in ((ka_ref, ks_ref), (va_ref, vs_ref)):
        rows = c_ref[0]
        l = jax.lax.dot_general(rows, q, (((1,), (1,)), ((), ()))) * scale
        lmax = jnp.max(l)
        e = jnp.exp(l - lmax)
        den = jnp.sum(e)
        num = jax.lax.dot_general(e, rows, (((0,), (0,)), ((), ())))
        pred = num / den
        s_ref[...] = jnp.mean((pred - q) ** 2)[None, None, None]


def _surprise(q3, Ka, Va):
    b, ng, d = Ka.shape
    scale = float(d) ** -0.5
    sca = jax.ShapeDtypeStruct((b, 1, 1), jnp.float32)
    sspec = pl.BlockSpec((1, 1, 1), lambda i: (i, 0, 0))
    big = pl.BlockSpec((1, ng, d), lambda i: (i, 0, 0))
    return pl.pallas_call(
        functools.partial(_surprise_body, scale=scale),
        grid=(b,),
        in_specs=[
            pl.BlockSpec((1, 1, d), lambda i: (i, 0, 0)),
            big, big,
        ],
        out_specs=[sspec, sspec],
        out_shape=[sca, sca],
        compiler_params=pltpu.CompilerParams(
            dimension_semantics=("arbitrary",)),
    )(q3, Ka, Va)


def _merge_body(ks_ref, vs_ref, mom_ref, idx_ref, wk_ref, bk_ref, wv_ref,
                bv_ref, eta_ref, alpha_ref, kc_ref, kp_ref, vc_ref, vp_ref,
                nm_ref, ko_ref, vo_ref, *, tm):
    mt = pl.program_id(1)
    ks = ks_ref[...]
    vs = vs_ref[...]
    alpha = jax.nn.sigmoid(alpha_ref[0, 0])
    comb = alpha * ks + (1.0 - alpha) * vs
    eta = jax.nn.sigmoid(eta_ref[0, 0])
    nm = eta * mom_ref[...] + (1.0 - eta) * comb

    @pl.when(mt == 0)
    def _():
        nm_ref[...] = nm

    idx = idx_ref[0]
    ng = idx.shape[-1]
    m_ids = mt * tm + jax.lax.broadcasted_iota(jnp.int32, (tm, 1), 0)
    sel = (idx == m_ids)
    n_iota = jax.lax.broadcasted_iota(jnp.int32, (tm, ng), 1)
    n_sel = jnp.max(jnp.where(sel, n_iota, -1), axis=1, keepdims=True)
    pick = sel & (n_iota == n_sel)
    active = n_sel >= 0

    for (s, w_ref, b_ref, c_ref, p_ref, o_ref) in (
            (ks, wk_ref, bk_ref, kc_ref, kp_ref, ko_ref),
            (vs, wv_ref, bv_ref, vc_ref, vp_ref, vo_ref)):
        gate = jax.nn.sigmoid(s[0] * w_ref[:, 0:1].T + nm[0] * w_ref[:, 1:2].T
                              + b_ref[...])
        g = jnp.sum(jnp.where(pick, gate, 0.0), axis=1, keepdims=True)
        g = jnp.where(active, g, 1.0)[None]
        o_ref[...] = g * c_ref[...] + (1.0 - g) * p_ref[...]


def _merge(ks, vs, mom3, idx3, Wk, bk2, Wv, bv2, eta2, alpha2,
           K_curr, K_prev, V_curr, V_prev, tm=256):
    b, m, d = K_curr.shape
    ng = idx3.shape[-1]
    sspec = pl.BlockSpec((1, 1, 1), lambda i, mt: (i, 0, 0))
    big = pl.BlockSpec((1, tm, d), lambda i, mt: (i, mt, 0))
    whole = lambda shape: pl.BlockSpec(
        shape, lambda i, mt, _s=shape: tuple(0 for _ in _s))
    out = jax.ShapeDtypeStruct((b, m, d), jnp.float32)
    return pl.pallas_call(
        functools.partial(_merge_body, tm=tm),
        grid=(b, m // tm),
        in_specs=[
            sspec, sspec, sspec,
            pl.BlockSpec((1, 1, ng), lambda i, mt: (i, 0, 0)),
            whole((ng, 2)), whole((1, ng)), whole((ng, 2)), whole((1, ng)),
            whole((1, 1)), whole((1, 1)),
            big, big, big, big,
        ],
        out_specs=[sspec, big, big],
        out_shape=[jax.ShapeDtypeStruct((b, 1, 1), jnp.float32), out, out],
        compiler_params=pltpu.CompilerParams(
            dimension_semantics=("parallel", "arbitrary")),
    )(ks, vs, mom3, idx3, Wk, bk2, Wv, bv2, eta2, alpha2,
      K_curr, K_prev, V_curr, V_prev)


def kernel(K_curr, V_curr, K_prev, V_prev, h, momentum, active_idx,
           Wk, bk, Wv, bv, logit_eta, surprise_logit_alpha):
    b, m, d = K_curr.shape
    ng = active_idx.shape[1]

    idx3 = active_idx.astype(jnp.int32).reshape(b, 1, ng)
    q3 = _qprobe(h)
    idx32 = active_idx.astype(jnp.int32)
    gidx = (idx32 + jnp.arange(b, dtype=jnp.int32)[:, None] * m).reshape(-1)
    Ka_flat, Va_flat = _sc_gather(
        K_curr.reshape(b * m, d), V_curr.reshape(b * m, d), gidx, d)
    ks, vs = _surprise(q3, Ka_flat.reshape(b, ng, d),
                       Va_flat.reshape(b, ng, d))
    nm3, K_out, V_out = _merge(
        ks, vs, momentum.reshape(b, 1, 1), idx3, Wk, bk.reshape(1, ng),
        Wv, bv.reshape(1, ng), jnp.reshape(logit_eta, (1, 1)),
        jnp.reshape(surprise_logit_alpha, (1, 1)),
        K_curr, K_prev, V_curr, V_prev)
    return (K_out, V_out, nm3.reshape(b, 1))


# speedup vs baseline: 1.3686x; 1.3686x over previous
import functools

import jax
import jax.numpy as jnp
from jax.experimental import pallas as pl
from jax.experimental.pallas import tpu as pltpu


_NEG = -1e30


def _surprise_body(h_ref, idx_ref, kc_ref, vc_ref, ks_ref, vs_ref, q_ref,
                   *, cm, scale):
    q_ref[...] = jnp.mean(h_ref[...], axis=1, keepdims=True)
    q = q_ref[0]
    idx = idx_ref[0]
    m = kc_ref.shape[1]
    mults = []
    for c in range(m // cm):
        m_ids = c * cm + jax.lax.broadcasted_iota(jnp.int32, (cm, 1), 0)
        sel = (idx == m_ids)
        mults.append(jnp.sum(jnp.where(sel, 1.0, 0.0), axis=1,
                             keepdims=True))
    mult = jnp.concatenate(mults, axis=0)
    has = mult > 0.0
    for (c_ref, s_ref) in ((kc_ref, ks_ref), (vc_ref, vs_ref)):
        rows = c_ref[0]
        l = jax.lax.dot_general(rows, q, (((1,), (1,)), ((), ()))) * scale
        lmax = jnp.max(jnp.where(has, l, _NEG))
        e = jnp.exp(l - lmax) * mult
        den = jnp.sum(e)
        num = jax.lax.dot_general(e, rows, (((0,), (0,)), ((), ())))
        pred = num / den
        s_ref[...] = jnp.mean((pred - q) ** 2)[None, None, None]


def _surprise(h, idx3, K_curr, V_curr, cm=512):
    b, m, d = K_curr.shape
    seq = h.shape[1]
    ng = idx3.shape[-1]
    scale = float(d) ** -0.5
    sca = jax.ShapeDtypeStruct((b, 1, 1), jnp.float32)
    sspec = pl.BlockSpec((1, 1, 1), lambda i: (i, 0, 0))
    big = pl.BlockSpec((1, m, d), lambda i: (i, 0, 0))
    return pl.pallas_call(
        functools.partial(_surprise_body, cm=cm, scale=scale),
        grid=(b,),
        in_specs=[
            pl.BlockSpec((1, seq, d), lambda i: (i, 0, 0)),
            pl.BlockSpec((1, 1, ng), lambda i: (i, 0, 0)),
            big, big,
        ],
        out_specs=[sspec, sspec, pl.BlockSpec((1, 1, d), lambda i: (i, 0, 0))],
        out_shape=[sca, sca, jax.ShapeDtypeStruct((b, 1, d), jnp.float32)],
        compiler_params=pltpu.CompilerParams(
            dimension_semantics=("arbitrary",)),
    )(h, idx3, K_curr, V_curr)


def _merge_body(ks_ref, vs_ref, mom_ref, idx_ref, wk_ref, bk_ref, wv_ref,
                bv_ref, eta_ref, alpha_ref, kc_ref, kp_ref, vc_ref, vp_ref,
                nm_ref, ko_ref, vo_ref, *, tm):
    mt = pl.program_id(1)
    ks = ks_ref[...]
    vs = vs_ref[...]
    alpha = jax.nn.sigmoid(alpha_ref[0, 0])
    comb = alpha * ks + (1.0 - alpha) * vs
    eta = jax.nn.sigmoid(eta_ref[0, 0])
    nm = eta * mom_ref[...] + (1.0 - eta) * comb

    @pl.when(mt == 0)
    def _():
        nm_ref[...] = nm

    idx = idx_ref[0]
    ng = idx.shape[-1]
    m_ids = mt * tm + jax.lax.broadcasted_iota(jnp.int32, (tm, 1), 0)
    sel = (idx == m_ids)
    n_iota = jax.lax.broadcasted_iota(jnp.int32, (tm, ng), 1)
    n_sel = jnp.max(jnp.where(sel, n_iota, -1), axis=1, keepdims=True)
    pick = sel & (n_iota == n_sel)
    active = n_sel >= 0

    for (s, w_ref, b_ref, c_ref, p_ref, o_ref) in (
            (ks, wk_ref, bk_ref, kc_ref, kp_ref, ko_ref),
            (vs, wv_ref, bv_ref, vc_ref, vp_ref, vo_ref)):
        gate = jax.nn.sigmoid(s[0] * w_ref[:, 0:1].T + nm[0] * w_ref[:, 1:2].T
                              + b_ref[...])
        g = jnp.sum(jnp.where(pick, gate, 0.0), axis=1, keepdims=True)
        g = jnp.where(active, g, 1.0)[None]
        o_ref[...] = g * c_ref[...] + (1.0 - g) * p_ref[...]


def _merge(ks, vs, mom3, idx3, Wk, bk2, Wv, bv2, eta2, alpha2,
           K_curr, K_prev, V_curr, V_prev, tm=512):
    b, m, d = K_curr.shape
    ng = idx3.shape[-1]
    sspec = pl.BlockSpec((1, 1, 1), lambda i, mt: (i, 0, 0))
    big = pl.BlockSpec((1, tm, d), lambda i, mt: (i, mt, 0))
    whole = lambda shape: pl.BlockSpec(
        shape, lambda i, mt, _s=shape: tuple(0 for _ in _s))
    out = jax.ShapeDtypeStruct((b, m, d), jnp.float32)
    return pl.pallas_call(
        functools.partial(_merge_body, tm=tm),
        grid=(b, m // tm),
        in_specs=[
            sspec, sspec, sspec,
            pl.BlockSpec((1, 1, ng), lambda i, mt: (i, 0, 0)),
            whole((ng, 2)), whole((1, ng)), whole((ng, 2)), whole((1, ng)),
            whole((1, 1)), whole((1, 1)),
            big, big, big, big,
        ],
        out_specs=[sspec, big, big],
        out_shape=[jax.ShapeDtypeStruct((b, 1, 1), jnp.float32), out, out],
        compiler_params=pltpu.CompilerParams(
            dimension_semantics=("parallel", "arbitrary")),
    )(ks, vs, mom3, idx3, Wk, bk2, Wv, bv2, eta2, alpha2,
      K_curr, K_prev, V_curr, V_prev)


def kernel(K_curr, V_curr, K_prev, V_prev, h, momentum, active_idx,
           Wk, bk, Wv, bv, logit_eta, surprise_logit_alpha):
    b, m, d = K_curr.shape
    ng = active_idx.shape[1]

    idx3 = active_idx.astype(jnp.int32).reshape(b, 1, ng)
    ks, vs, _ = _surprise(h, idx3, K_curr, V_curr)
    nm3, K_out, V_out = _merge(
        ks, vs, momentum.reshape(b, 1, 1), idx3, Wk, bk.reshape(1, ng),
        Wv, bv.reshape(1, ng), jnp.reshape(logit_eta, (1, 1)),
        jnp.reshape(surprise_logit_alpha, (1, 1)),
        K_curr, K_prev, V_curr, V_prev)
    return (K_out, V_out, nm3.reshape(b, 1))


# speedup vs baseline: 1.3690x; 1.0003x over previous
import functools

import jax
import jax.numpy as jnp
from jax.experimental import pallas as pl
from jax.experimental.pallas import tpu as pltpu


_NEG = -1e30


def _surprise_body(h_ref, idx_ref, kc_ref, vc_ref, ks_ref, vs_ref, q_ref,
                   *, cm, scale):
    q_ref[...] = jnp.mean(h_ref[...], axis=1, keepdims=True)
    q = q_ref[0]
    idx = idx_ref[0]
    m = kc_ref.shape[1]
    mults = []
    for c in range(m // cm):
        m_ids = c * cm + jax.lax.broadcasted_iota(jnp.int32, (cm, 1), 0)
        sel = (idx == m_ids)
        mults.append(jnp.sum(jnp.where(sel, 1.0, 0.0), axis=1,
                             keepdims=True))
    mult = jnp.concatenate(mults, axis=0)
    has = mult > 0.0
    for (c_ref, s_ref) in ((kc_ref, ks_ref), (vc_ref, vs_ref)):
        rows = c_ref[0]
        l = jax.lax.dot_general(rows, q, (((1,), (1,)), ((), ()))) * scale
        lmax = jnp.max(jnp.where(has, l, _NEG))
        e = jnp.exp(l - lmax) * mult
        den = jnp.sum(e)
        num = jax.lax.dot_general(e, rows, (((0,), (0,)), ((), ())))
        pred = num / den
        s_ref[...] = jnp.mean((pred - q) ** 2)[None, None, None]


def _surprise(h, idx3, K_curr, V_curr, cm=512):
    b, m, d = K_curr.shape
    seq = h.shape[1]
    ng = idx3.shape[-1]
    scale = float(d) ** -0.5
    sca = jax.ShapeDtypeStruct((b, 1, 1), jnp.float32)
    sspec = pl.BlockSpec((1, 1, 1), lambda i: (i, 0, 0))
    big = pl.BlockSpec((1, m, d), lambda i: (i, 0, 0))
    return pl.pallas_call(
        functools.partial(_surprise_body, cm=cm, scale=scale),
        grid=(b,),
        in_specs=[
            pl.BlockSpec((1, seq, d), lambda i: (i, 0, 0)),
            pl.BlockSpec((1, 1, ng), lambda i: (i, 0, 0)),
            big, big,
        ],
        out_specs=[sspec, sspec, pl.BlockSpec((1, 1, d), lambda i: (i, 0, 0))],
        out_shape=[sca, sca, jax.ShapeDtypeStruct((b, 1, d), jnp.float32)],
        compiler_params=pltpu.CompilerParams(
            dimension_semantics=("arbitrary",)),
    )(h, idx3, K_curr, V_curr)


def _merge_body(ks_ref, vs_ref, mom_ref, idx_ref, wk_ref, bk_ref, wv_ref,
                bv_ref, eta_ref, alpha_ref, kc_ref, kp_ref, vc_ref, vp_ref,
                nm_ref, ko_ref, vo_ref, *, tm):
    mt = pl.program_id(1)
    ks = ks_ref[...]
    vs = vs_ref[...]
    alpha = jax.nn.sigmoid(alpha_ref[0, 0])
    comb = alpha * ks + (1.0 - alpha) * vs
    eta = jax.nn.sigmoid(eta_ref[0, 0])
    nm = eta * mom_ref[...] + (1.0 - eta) * comb

    @pl.when(mt == 0)
    def _():
        nm_ref[...] = nm

    idx = idx_ref[0]
    ng = idx.shape[-1]
    m_ids = mt * tm + jax.lax.broadcasted_iota(jnp.int32, (tm, 1), 0)
    sel = (idx == m_ids)
    n_iota = jax.lax.broadcasted_iota(jnp.int32, (tm, ng), 1)
    n_sel = jnp.max(jnp.where(sel, n_iota, -1), axis=1, keepdims=True)
    pick = sel & (n_iota == n_sel)
    active = n_sel >= 0

    for (s, w_ref, b_ref, c_ref, p_ref, o_ref) in (
            (ks, wk_ref, bk_ref, kc_ref, kp_ref, ko_ref),
            (vs, wv_ref, bv_ref, vc_ref, vp_ref, vo_ref)):
        gate = jax.nn.sigmoid(s[0] * w_ref[:, 0:1].T + nm[0] * w_ref[:, 1:2].T
                              + b_ref[...])
        g = jnp.sum(jnp.where(pick, gate, 0.0), axis=1, keepdims=True)
        g = jnp.where(active, g, 1.0)[None]
        o_ref[...] = g * c_ref[...] + (1.0 - g) * p_ref[...]


def _merge(ks, vs, mom3, idx3, Wk, bk2, Wv, bv2, eta2, alpha2,
           K_curr, K_prev, V_curr, V_prev, tm=1024):
    b, m, d = K_curr.shape
    ng = idx3.shape[-1]
    sspec = pl.BlockSpec((1, 1, 1), lambda i, mt: (i, 0, 0))
    big = pl.BlockSpec((1, tm, d), lambda i, mt: (i, mt, 0))
    whole = lambda shape: pl.BlockSpec(
        shape, lambda i, mt, _s=shape: tuple(0 for _ in _s))
    out = jax.ShapeDtypeStruct((b, m, d), jnp.float32)
    return pl.pallas_call(
        functools.partial(_merge_body, tm=tm),
        grid=(b, m // tm),
        in_specs=[
            sspec, sspec, sspec,
            pl.BlockSpec((1, 1, ng), lambda i, mt: (i, 0, 0)),
            whole((ng, 2)), whole((1, ng)), whole((ng, 2)), whole((1, ng)),
            whole((1, 1)), whole((1, 1)),
            big, big, big, big,
        ],
        out_specs=[sspec, big, big],
        out_shape=[jax.ShapeDtypeStruct((b, 1, 1), jnp.float32), out, out],
        compiler_params=pltpu.CompilerParams(
            dimension_semantics=("parallel", "arbitrary")),
    )(ks, vs, mom3, idx3, Wk, bk2, Wv, bv2, eta2, alpha2,
      K_curr, K_prev, V_curr, V_prev)


def kernel(K_curr, V_curr, K_prev, V_prev, h, momentum, active_idx,
           Wk, bk, Wv, bv, logit_eta, surprise_logit_alpha):
    b, m, d = K_curr.shape
    ng = active_idx.shape[1]

    idx3 = active_idx.astype(jnp.int32).reshape(b, 1, ng)
    ks, vs, _ = _surprise(h, idx3, K_curr, V_curr)
    nm3, K_out, V_out = _merge(
        ks, vs, momentum.reshape(b, 1, 1), idx3, Wk, bk.reshape(1, ng),
        Wv, bv.reshape(1, ng), jnp.reshape(logit_eta, (1, 1)),
        jnp.reshape(surprise_logit_alpha, (1, 1)),
        K_curr, K_prev, V_curr, V_prev)
    return (K_out, V_out, nm3.reshape(b, 1))
